# baseline (speedup 1.0000x reference)
import functools

import jax
import jax.numpy as jnp
from jax import lax
from jax.experimental import pallas as pl
from jax.experimental.pallas import tpu as pltpu

_NA = 256


def _full_body(h_ref, s_ref, x_ref, o_ref, S_scratch, *, d_in, d_out):
    @pl.when(pl.program_id(0) == 0)
    def _build():
        col = lax.broadcasted_iota(jnp.int32, (d_in, d_out), 1)
        for i in range(3):
            h = h_ref[i, :].reshape(d_in, 1)
            sg = s_ref[i, :].reshape(d_in, 1).astype(jnp.float32)
            S_scratch[i] = jnp.where(col == h, sg, 0.0)

    x = x_ref[...]
    acc = jnp.dot(x, S_scratch[0], preferred_element_type=jnp.float32)
    acc = acc * jnp.dot(x, S_scratch[1], preferred_element_type=jnp.float32)
    acc = acc * jnp.dot(x, S_scratch[2], preferred_element_type=jnp.float32)
    o_ref[...] = acc


def _build_body(h_ref, s_ref, g_ref, e_ref, *, d_in, d_out):
    col = lax.broadcasted_iota(jnp.int32, (d_in, d_out), 1)
    pres = []
    for i in range(3):
        m = col == h_ref[i, :].reshape(d_in, 1)
        pres.append(jnp.max(m.astype(jnp.int32), axis=0, keepdims=True))
    active = pres[0] * pres[1] * pres[2]
    r0 = lax.broadcasted_iota(jnp.int32, (d_out, d_out), 0)
    c0 = lax.broadcasted_iota(jnp.int32, (d_out, d_out), 1)
    tri = (r0 <= c0).astype(jnp.float32)
    rank = (
        jnp.dot(active.astype(jnp.float32), tri, preferred_element_type=jnp.float32)
        .astype(jnp.int32)
        - 1
    )

    colA = lax.broadcasted_iota(jnp.int32, (d_in, _NA), 1)
    for i in range(3):
        m = col == h_ref[i, :].reshape(d_in, 1)
        mi = m.astype(jnp.int32)
        valid = jnp.sum(mi * active, axis=1, keepdims=True)
        slot = jnp.sum(mi * rank, axis=1, keepdims=True)
        slot = jnp.where(valid > 0, slot, _NA)
        sg = s_ref[i, :].reshape(d_in, 1).astype(jnp.float32)
        g_ref[:, i * _NA:(i + 1) * _NA] = jnp.where(colA == slot, sg, 0.0)

    rowA = lax.broadcasted_iota(jnp.int32, (_NA, d_out), 0)
    e_ref[...] = jnp.where((rowA == rank) & (active > 0), 1.0, 0.0)


def _main_body(g_ref, e_ref, x_ref, o_ref):
    x = x_ref[...]
    y = jnp.dot(x, g_ref[...], preferred_element_type=jnp.float32)
    p = y[:, :_NA] * y[:, _NA:2 * _NA] * y[:, 2 * _NA:3 * _NA]
    o_ref[...] = jnp.dot(p, e_ref[...], preferred_element_type=jnp.float32)


@functools.partial(jax.jit, static_argnames=("tile",))
def _run(x, hash_indices, signs, tile=1024):
    batch, d_in = x.shape
    d_out = min(2048, 2 * d_in)

    build_call = pl.pallas_call(
        functools.partial(_build_body, d_in=d_in, d_out=d_out),
        grid=(1,),
        in_specs=[
            pl.BlockSpec((3, d_in), lambda i: (0, 0)),
            pl.BlockSpec((3, d_in), lambda i: (0, 0)),
        ],
        out_specs=[
            pl.BlockSpec((d_in, 3 * _NA), lambda i: (0, 0)),
            pl.BlockSpec((_NA, d_out), lambda i: (0, 0)),
        ],
        out_shape=[
            jax.ShapeDtypeStruct((d_in, 3 * _NA), jnp.float32),
            jax.ShapeDtypeStruct((_NA, d_out), jnp.float32),
        ],
    )

    main_call = pl.pallas_call(
        _main_body,
        grid=(batch // tile,),
        in_specs=[
            pl.BlockSpec((d_in, 3 * _NA), lambda i: (0, 0)),
            pl.BlockSpec((_NA, d_out), lambda i: (0, 0)),
            pl.BlockSpec((tile, d_in), lambda i: (i, 0)),
        ],
        out_specs=pl.BlockSpec((tile, d_out), lambda i: (i, 0)),
        out_shape=jax.ShapeDtypeStruct((batch, d_out), jnp.float32),
    )

    full_call = pl.pallas_call(
        functools.partial(_full_body, d_in=d_in, d_out=d_out),
        grid=(batch // 512,),
        in_specs=[
            pl.BlockSpec((3, d_in), lambda i: (0, 0)),
            pl.BlockSpec((3, d_in), lambda i: (0, 0)),
            pl.BlockSpec((512, d_in), lambda i: (i, 0)),
        ],
        out_specs=pl.BlockSpec((512, d_out), lambda i: (i, 0)),
        out_shape=jax.ShapeDtypeStruct((batch, d_out), jnp.float32),
        scratch_shapes=[pltpu.VMEM((3, d_in, d_out), jnp.float32)],
    )

    pres = [
        jnp.zeros((d_out,), jnp.bool_).at[hash_indices[i]].set(True, mode="drop")
        for i in range(3)
    ]
    n_active = jnp.sum(pres[0] & pres[1] & pres[2])

    def compact_path():
        G, E = build_call(hash_indices, signs)
        return main_call(G, E, x)

    return lax.cond(
        n_active <= _NA,
        compact_path,
        lambda: full_call(hash_indices, signs, x),
    )


def kernel(x, hash_indices, signs):
    return _run(x, hash_indices, signs)


# speedup vs baseline: 1.0397x; 1.0397x over previous
import functools

import jax
import jax.numpy as jnp
from jax import lax
from jax.experimental import pallas as pl
from jax.experimental.pallas import tpu as pltpu

_NA = 256


def _full_body(h_ref, s_ref, x_ref, o_ref, S_scratch, *, d_in, d_out):
    @pl.when(pl.program_id(0) == 0)
    def _build():
        col = lax.broadcasted_iota(jnp.int32, (d_in, d_out), 1)
        for i in range(3):
            h = h_ref[i, :].reshape(d_in, 1)
            sg = s_ref[i, :].reshape(d_in, 1).astype(jnp.float32)
            S_scratch[i] = jnp.where(col == h, sg, 0.0)

    x = x_ref[...]
    acc = jnp.dot(x, S_scratch[0], preferred_element_type=jnp.float32)
    acc = acc * jnp.dot(x, S_scratch[1], preferred_element_type=jnp.float32)
    acc = acc * jnp.dot(x, S_scratch[2], preferred_element_type=jnp.float32)
    o_ref[...] = acc


def _compact_body(h_ref, s_ref, x_ref, o_ref, G0_ref, G1_ref, G2_ref, E_ref, *, d_in, d_out):
    G_refs = (G0_ref, G1_ref, G2_ref)

    @pl.when(pl.program_id(0) == 0)
    def _build():
        col = lax.broadcasted_iota(jnp.int32, (d_in, d_out), 1)
        pres = []
        for i in range(3):
            m = col == h_ref[i, :].reshape(d_in, 1)
            pres.append(jnp.max(m.astype(jnp.int32), axis=0, keepdims=True))
        active = pres[0] * pres[1] * pres[2]
        r0 = lax.broadcasted_iota(jnp.int32, (d_out, d_out), 0)
        c0 = lax.broadcasted_iota(jnp.int32, (d_out, d_out), 1)
        tri = (r0 <= c0).astype(jnp.float32)
        rank = (
            jnp.dot(active.astype(jnp.float32), tri, preferred_element_type=jnp.float32)
            .astype(jnp.int32)
            - 1
        )

        colA = lax.broadcasted_iota(jnp.int32, (d_in, _NA), 1)
        for i in range(3):
            m = col == h_ref[i, :].reshape(d_in, 1)
            mi = m.astype(jnp.int32)
            valid = jnp.sum(mi * active, axis=1, keepdims=True)
            slot = jnp.sum(mi * rank, axis=1, keepdims=True)
            slot = jnp.where(valid > 0, slot, _NA)
            sg = s_ref[i, :].reshape(d_in, 1).astype(jnp.float32)
            G_refs[i][...] = jnp.where(colA == slot, sg, 0.0)

        rowA = lax.broadcasted_iota(jnp.int32, (_NA, d_out), 0)
        E_ref[...] = jnp.where((rowA == rank) & (active > 0), 1.0, 0.0)

    x = x_ref[...]
    y = jnp.dot(x, G0_ref[...], preferred_element_type=jnp.float32)
    y = y * jnp.dot(x, G1_ref[...], preferred_element_type=jnp.float32)
    y = y * jnp.dot(x, G2_ref[...], preferred_element_type=jnp.float32)
    o_ref[...] = jnp.dot(y, E_ref[...], preferred_element_type=jnp.float32)


def _make_call(body, scratch_shapes, d_in, d_out, batch, tile):
    return pl.pallas_call(
        functools.partial(body, d_in=d_in, d_out=d_out),
        grid=(batch // tile,),
        in_specs=[
            pl.BlockSpec((3, d_in), lambda i: (0, 0)),
            pl.BlockSpec((3, d_in), lambda i: (0, 0)),
            pl.BlockSpec((tile, d_in), lambda i: (i, 0)),
        ],
        out_specs=pl.BlockSpec((tile, d_out), lambda i: (i, 0)),
        out_shape=jax.ShapeDtypeStruct((batch, d_out), jnp.float32),
        scratch_shapes=scratch_shapes,
    )


@functools.partial(jax.jit, static_argnames=("tile",))
def _run(x, hash_indices, signs, tile=1024):
    batch, d_in = x.shape
    d_out = min(2048, 2 * d_in)

    compact_call = _make_call(
        _compact_body,
        [
            pltpu.VMEM((d_in, _NA), jnp.float32),
            pltpu.VMEM((d_in, _NA), jnp.float32),
            pltpu.VMEM((d_in, _NA), jnp.float32),
            pltpu.VMEM((_NA, d_out), jnp.float32),
        ],
        d_in, d_out, batch, tile,
    )
    full_call = _make_call(
        _full_body,
        [pltpu.VMEM((3, d_in, d_out), jnp.float32)],
        d_in, d_out, batch, min(tile, 512),
    )

    pres = [
        jnp.zeros((d_out,), jnp.bool_).at[hash_indices[i]].set(True, mode="drop")
        for i in range(3)
    ]
    n_active = jnp.sum(pres[0] & pres[1] & pres[2])

    return lax.cond(
        n_active <= _NA,
        lambda: compact_call(hash_indices, signs, x),
        lambda: full_call(hash_indices, signs, x),
    )


def kernel(x, hash_indices, signs):
    return _run(x, hash_indices, signs)
